# baseline (speedup 1.0000x reference)
import functools

import jax
import jax.numpy as jnp
from jax import lax
from jax.experimental import pallas as pl
from jax.experimental.pallas import tpu as pltpu
from jax.experimental.pallas import tpu_sc as plsc

SEQ = 16384
Q = 32
D = 4 * Q
NC = 2
NS = 16
NW = NC * NS
BPW = SEQ // NW
L = 16

_TAB_ROWS = (24, 7, 12, 10)

_mesh = plsc.VectorSubcoreMesh(core_axis_name="c", subcore_axis_name="s")


@functools.partial(
    pl.kernel,
    out_type=jax.ShapeDtypeStruct((SEQ, D), jnp.float32),
    mesh=_mesh,
    compiler_params=pltpu.CompilerParams(needs_layout_passes=False),
    scratch_types=[
        pltpu.VMEM((BPW,), jnp.int32),
        pltpu.VMEM((BPW,), jnp.int32),
        pltpu.VMEM((BPW,), jnp.int32),
        pltpu.VMEM((BPW,), jnp.int32),
        pltpu.VMEM((_TAB_ROWS[0], Q), jnp.float32),
        pltpu.VMEM((_TAB_ROWS[1], Q), jnp.float32),
        pltpu.VMEM((_TAB_ROWS[2], Q), jnp.float32),
        pltpu.VMEM((_TAB_ROWS[3], Q), jnp.float32),
        pltpu.VMEM((BPW, D), jnp.float32),
        pltpu.SemaphoreType.DMA,
    ],
)
def _encode(hours, days, months, years, wh, wd, wm, wy, out,
            ih_v, id_v, im_v, iy_v, th_v, td_v, tm_v, ty_v, out_v, sem):
    wid = lax.axis_index("s") * NC + lax.axis_index("c")
    base = wid * BPW

    for src, dst in ((wh, th_v), (wd, td_v), (wm, tm_v), (wy, ty_v)):
        pltpu.sync_copy(src, dst)
    idx_vs = (ih_v, id_v, im_v, iy_v)
    for src, dst in zip((hours, days, months, years), idx_vs):
        pltpu.sync_copy(src.at[pl.ds(base, BPW)], dst)

    tab_vs = (th_v, td_v, tm_v, ty_v)

    n_chunks = 4
    gpc = (BPW // L) // n_chunks
    rpc = BPW // n_chunks
    copies = []
    for ch in range(n_chunks):
        @plsc.parallel_loop(ch * gpc, (ch + 1) * gpc, unroll=2)
        def body(g):
            row0 = g * L
            vs = [idx_vs[t][pl.ds(row0, L)] for t in range(4)]
            for j in range(L):
                r = row0 + j
                for t in range(4):
                    s = vs[t][j]
                    for c0 in range(0, Q, L):
                        out_v[r, pl.ds(t * Q + c0, L)] = \
                            tab_vs[t][s, pl.ds(c0, L)]

        copies.append(
            pltpu.async_copy(out_v.at[pl.ds(ch * rpc, rpc), :],
                             out.at[pl.ds(base + ch * rpc, rpc), :], sem))
    for c in copies:
        c.wait()


def kernel(hours, days, months, years, W_hour, W_day, W_month, W_year):
    return _encode(hours.astype(jnp.int32), days.astype(jnp.int32),
                   months.astype(jnp.int32), years.astype(jnp.int32),
                   W_hour, W_day, W_month, W_year)


# speedup vs baseline: 1.2623x; 1.2623x over previous
import functools

import jax
import jax.numpy as jnp
from jax import lax
from jax.experimental import pallas as pl
from jax.experimental.pallas import tpu as pltpu
from jax.experimental.pallas import tpu_sc as plsc

SEQ = 16384
Q = 32
D = 4 * Q
NC = 2
NS = 16
NW = NC * NS
BPW = SEQ // NW
L = 16

_TAB_ROWS = (24, 7, 12, 10)

_mesh = plsc.VectorSubcoreMesh(core_axis_name="c", subcore_axis_name="s")


@functools.partial(
    pl.kernel,
    out_type=jax.ShapeDtypeStruct((SEQ, D), jnp.float32),
    mesh=_mesh,
    compiler_params=pltpu.CompilerParams(needs_layout_passes=False),
    scratch_types=[
        pltpu.VMEM((BPW,), jnp.int32),
        pltpu.VMEM((BPW,), jnp.int32),
        pltpu.VMEM((BPW,), jnp.int32),
        pltpu.VMEM((BPW,), jnp.int32),
        pltpu.VMEM((_TAB_ROWS[0], Q), jnp.float32),
        pltpu.VMEM((_TAB_ROWS[1], Q), jnp.float32),
        pltpu.VMEM((_TAB_ROWS[2], Q), jnp.float32),
        pltpu.VMEM((_TAB_ROWS[3], Q), jnp.float32),
        pltpu.VMEM((BPW, D), jnp.float32),
    ],
)
def _encode(hours, days, months, years, wh, wd, wm, wy, out,
            ih_v, id_v, im_v, iy_v, th_v, td_v, tm_v, ty_v, out_v):
    wid = lax.axis_index("s") * NC + lax.axis_index("c")
    base = wid * BPW

    for src, dst in ((wh, th_v), (wd, td_v), (wm, tm_v), (wy, ty_v)):
        pltpu.sync_copy(src, dst)
    idx_vs = (ih_v, id_v, im_v, iy_v)
    for src, dst in zip((hours, days, months, years), idx_vs):
        pltpu.sync_copy(src.at[pl.ds(base, BPW)], dst)

    tab_vs = (th_v, td_v, tm_v, ty_v)

    @plsc.parallel_loop(0, BPW // L, unroll=1)
    def body(g):
        row0 = g * L
        vs = [idx_vs[t][pl.ds(row0, L)] for t in range(4)]
        for j in range(L):
            r = row0 + j
            for t in range(4):
                s = vs[t][j]
                for c0 in range(0, Q, L):
                    out_v[r, pl.ds(t * Q + c0, L)] = tab_vs[t][s, pl.ds(c0, L)]

    pltpu.sync_copy(out_v, out.at[pl.ds(base, BPW), :])


def kernel(hours, days, months, years, W_hour, W_day, W_month, W_year):
    return _encode(hours.astype(jnp.int32), days.astype(jnp.int32),
                   months.astype(jnp.int32), years.astype(jnp.int32),
                   W_hour, W_day, W_month, W_year)


# speedup vs baseline: 1.3984x; 1.1078x over previous
import functools

import jax
import jax.numpy as jnp
from jax import lax
from jax.experimental import pallas as pl
from jax.experimental.pallas import tpu as pltpu
from jax.experimental.pallas import tpu_sc as plsc

SEQ = 16384
Q = 32
D = 4 * Q
NC = 2
NS = 16
NW = NC * NS
BPW = SEQ // NW
L = 16

_TAB_ROWS = (24, 7, 12, 10)

_mesh = plsc.VectorSubcoreMesh(core_axis_name="c", subcore_axis_name="s")


@functools.partial(
    pl.kernel,
    out_type=jax.ShapeDtypeStruct((SEQ, D), jnp.float32),
    mesh=_mesh,
    compiler_params=pltpu.CompilerParams(needs_layout_passes=False),
    scratch_types=[
        pltpu.VMEM((BPW,), jnp.int32),
        pltpu.VMEM((BPW,), jnp.int32),
        pltpu.VMEM((BPW,), jnp.int32),
        pltpu.VMEM((BPW,), jnp.int32),
        pltpu.VMEM((_TAB_ROWS[0], Q), jnp.float32),
        pltpu.VMEM((_TAB_ROWS[1], Q), jnp.float32),
        pltpu.VMEM((_TAB_ROWS[2], Q), jnp.float32),
        pltpu.VMEM((_TAB_ROWS[3], Q), jnp.float32),
        pltpu.VMEM((BPW, D), jnp.float32),
        pltpu.SemaphoreType.DMA,
    ],
)
def _encode(hours, days, months, years, wh, wd, wm, wy, out,
            ih_v, id_v, im_v, iy_v, th_v, td_v, tm_v, ty_v, out_v, sem):
    wid = lax.axis_index("s") * NC + lax.axis_index("c")
    base = wid * BPW

    idx_vs = (ih_v, id_v, im_v, iy_v)
    copies = [pltpu.async_copy(src, dst, sem)
              for src, dst in ((wh, th_v), (wd, td_v), (wm, tm_v),
                               (wy, ty_v))]
    copies += [pltpu.async_copy(src.at[pl.ds(base, BPW)], dst, sem)
               for src, dst in zip((hours, days, months, years), idx_vs)]
    for c in copies:
        c.wait()

    tab_vs = (th_v, td_v, tm_v, ty_v)

    @plsc.parallel_loop(0, BPW // L, unroll=1)
    def body(g):
        row0 = g * L
        vs = [idx_vs[t][pl.ds(row0, L)] for t in range(4)]
        for j in range(L):
            r = row0 + j
            for t in range(4):
                s = vs[t][j]
                for c0 in range(0, Q, L):
                    out_v[r, pl.ds(t * Q + c0, L)] = tab_vs[t][s, pl.ds(c0, L)]

    pltpu.sync_copy(out_v, out.at[pl.ds(base, BPW), :])


def kernel(hours, days, months, years, W_hour, W_day, W_month, W_year):
    return _encode(hours.astype(jnp.int32), days.astype(jnp.int32),
                   months.astype(jnp.int32), years.astype(jnp.int32),
                   W_hour, W_day, W_month, W_year)
